# baseline (speedup 1.0000x reference)
import jax
import jax.numpy as jnp
from jax import lax
from jax.experimental import pallas as pl
from jax.experimental.pallas import tpu as pltpu
from jax.experimental.pallas import tpu_sc as plsc

NJ = 50
NB = 16384
D = 32
NC, NS = 2, 16
NW = NC * NS
G = 4
CR = G * 128
NCBG = NB // CR
NBLK = NJ * NCBG
BPW = NBLK // NW
TW = G * 1024
NBUF = 2


def _gather_kernel(idx_hbm, table_hbm, out_hbm, idx_v, rows_v, t5_v, *sems):
    sem_i = sems[:NBUF]
    sem_g = sems[NBUF:2 * NBUF]
    sem_w = sems[2 * NBUF:]
    wid = lax.axis_index("s") * NC + lax.axis_index("c")
    t0 = wid * BPW

    iota = lax.iota(jnp.int32, 16)
    fbase = (iota // 8) * TW + (iota % 8) * 128
    fbase2 = fbase + 2 * TW

    def idx_desc(t, b):
        j = t // NCBG
        cbg = t % NCBG
        return pltpu.make_async_copy(
            idx_hbm.at[j, pl.ds(cbg * CR, CR)], idx_v.at[b], sem_i[b])

    def gather_desc(b):
        return pltpu.make_async_copy(
            table_hbm.at[idx_v.at[b]], rows_v.at[b], sem_g[b])

    def wb_descs(t, b):
        j = t // NCBG
        cbg = t % NCBG
        return [pltpu.make_async_copy(
                    t5_v.at[b, pl.ds(r * TW, TW)],
                    out_hbm.at[j, r, pl.ds(cbg * TW, TW)],
                    sem_w[b])
                for r in range(4)]

    def transpose_block(b):
        def sub(s, carry):
            base = s * 64
            for r in range(64):
                lb = base + r
                g_off = (lb // 128) * 1024 + (lb % 128)
                va = rows_v[b, lb, pl.ds(0, 16)]
                vb = rows_v[b, lb, pl.ds(16, 16)]
                plsc.store_scatter(t5_v.at[b], [fbase + g_off], va)
                plsc.store_scatter(t5_v.at[b], [fbase2 + g_off], vb)
            return carry

        lax.fori_loop(0, CR // 64, sub, 0)

    idx_desc(t0, 0).start()
    idx_desc(t0, 0).wait()
    gather_desc(0).start()
    idx_desc(t0 + 1, 1).start()

    def group(g_, carry):
        for b in range(NBUF):
            t = g_ * NBUF + b
            b1 = 1 - b

            @pl.when(t + 1 < BPW)
            def _():
                idx_desc(t0 + t + 1, b1).wait()
                gather_desc(b1).start()

            gather_desc(b).wait()

            @pl.when(t + 2 < BPW)
            def _():
                idx_desc(t0 + t + 2, b).start()

            @pl.when(t >= NBUF)
            def _():
                for d_ in wb_descs(t0 + t, b):
                    d_.wait()

            transpose_block(b)
            for d_ in wb_descs(t0 + t, b):
                d_.start()
        return carry

    lax.fori_loop(0, BPW // NBUF, group, 0)
    for b in range(NBUF):
        for d_ in wb_descs(t0 + BPW - NBUF + b, b):
            d_.wait()


@jax.jit
def _gather(idxT, table):
    mesh = plsc.VectorSubcoreMesh(core_axis_name="c", subcore_axis_name="s")
    f = pl.kernel(
        _gather_kernel,
        out_type=jax.ShapeDtypeStruct((NJ, 4, NB * 8), jnp.float32),
        mesh=mesh,
        scratch_types=(
            [pltpu.VMEM((NBUF, CR), jnp.int32),
             pltpu.VMEM((NBUF, CR, D), jnp.float32),
             pltpu.VMEM((NBUF, 4 * TW), jnp.float32)]
            + [pltpu.SemaphoreType.DMA] * (3 * NBUF)
        ),
        compiler_params=pltpu.CompilerParams(use_tc_tiling_on_sc=False,
                                             needs_layout_passes=False),
    )
    return f(idxT, table)


def kernel(input, table):
    out5 = _gather(input.T, table)
    return (out5.reshape(NJ, 4, NB // 128, 8, 128)
            .transpose(2, 4, 0, 1, 3)
            .reshape(NB, NJ, D))


# speedup vs baseline: 1.3860x; 1.3860x over previous
import jax
import jax.numpy as jnp
from jax import lax
from jax.experimental import pallas as pl
from jax.experimental.pallas import tpu as pltpu
from jax.experimental.pallas import tpu_sc as plsc

NJ = 50
NB = 16384
D = 32
NC, NS = 2, 16
NW = NC * NS
G = 4
CR = G * 128
NCBG = NB // CR
NBLK = NJ * NCBG
BPW = NBLK // NW
TW = G * 1024
NBUF = 2


def _gather_kernel(idx_hbm, table_hbm, out_hbm, idx_v, rows_v, t5_v, *sems):
    sem_i = sems[:NBUF]
    sem_g = sems[NBUF:2 * NBUF]
    sem_w = sems[2 * NBUF:]
    wid = lax.axis_index("s") * NC + lax.axis_index("c")
    t0 = wid * BPW

    iota = lax.iota(jnp.int32, 16)
    mvecs = [(iota + k) & 15 for k in range(16)]
    msvecs = [(mv >> 3) * 4096 + (mv & 7) * 128 for mv in mvecs]

    def idx_desc(t, b):
        j = t // NCBG
        cbg = t % NCBG
        return pltpu.make_async_copy(
            idx_hbm.at[j, pl.ds(cbg * CR, CR)], idx_v.at[b], sem_i[b])

    def gather_desc(b):
        return pltpu.make_async_copy(
            table_hbm.at[idx_v.at[b]], rows_v.at[b], sem_g[b])

    def wb_descs(t, b):
        j = t // NCBG
        cbg = t % NCBG
        return [pltpu.make_async_copy(
                    t5_v.at[b, pl.ds(r * TW, TW)],
                    out_hbm.at[j, r, pl.ds(cbg * TW, TW)],
                    sem_w[b])
                for r in range(4)]

    def transpose_block(b):
        def sub(ti, carry):
            lb0 = ti * 16
            riota = iota + lb0
            for cb2 in (0, 16):
                s_sc = (lb0 // 128) * 1024 + (lb0 % 128) + (cb2 // 8) * TW
                sbase = iota + s_sc
                for k in range(16):
                    cv = mvecs[k] if cb2 == 0 else mvecs[k] + cb2
                    vv = plsc.load_gather(rows_v.at[b], [riota, cv])
                    plsc.store_scatter(t5_v.at[b], [msvecs[k] + sbase], vv)
            return carry

        lax.fori_loop(0, CR // 16, sub, 0)

    idx_desc(t0, 0).start()
    idx_desc(t0, 0).wait()
    gather_desc(0).start()
    idx_desc(t0 + 1, 1).start()

    def group(g_, carry):
        for b in range(NBUF):
            t = g_ * NBUF + b
            b1 = 1 - b

            @pl.when(t + 1 < BPW)
            def _():
                idx_desc(t0 + t + 1, b1).wait()
                gather_desc(b1).start()

            gather_desc(b).wait()

            @pl.when(t + 2 < BPW)
            def _():
                idx_desc(t0 + t + 2, b).start()

            @pl.when(t >= NBUF)
            def _():
                for d_ in wb_descs(t0 + t, b):
                    d_.wait()

            transpose_block(b)
            for d_ in wb_descs(t0 + t, b):
                d_.start()
        return carry

    lax.fori_loop(0, BPW // NBUF, group, 0)
    for b in range(NBUF):
        for d_ in wb_descs(t0 + BPW - NBUF + b, b):
            d_.wait()


@jax.jit
def _gather(idxT, table):
    mesh = plsc.VectorSubcoreMesh(core_axis_name="c", subcore_axis_name="s")
    f = pl.kernel(
        _gather_kernel,
        out_type=jax.ShapeDtypeStruct((NJ, 4, NB * 8), jnp.float32),
        mesh=mesh,
        scratch_types=(
            [pltpu.VMEM((NBUF, CR), jnp.int32),
             pltpu.VMEM((NBUF, CR, D), jnp.float32),
             pltpu.VMEM((NBUF, 4 * TW), jnp.float32)]
            + [pltpu.SemaphoreType.DMA] * (3 * NBUF)
        ),
        compiler_params=pltpu.CompilerParams(use_tc_tiling_on_sc=False,
                                             needs_layout_passes=False),
    )
    return f(idxT, table)


def kernel(input, table):
    out5 = _gather(input.T, table)
    return (out5.reshape(NJ, 4, NB // 128, 8, 128)
            .transpose(2, 4, 0, 1, 3)
            .reshape(NB, NJ, D))


# speedup vs baseline: 2.0332x; 1.4669x over previous
import jax
import jax.numpy as jnp
from jax import lax
from jax.experimental import pallas as pl
from jax.experimental.pallas import tpu as pltpu
from jax.experimental.pallas import tpu_sc as plsc

NJ = 50
NB = 16384
D = 32
NC, NS = 2, 16
NW = NC * NS
G = 4
CR = G * 128
NCBG = NB // CR
NBLK = NJ * NCBG
BPW = NBLK // NW
TW = G * 1024
NBUF = 2


def _gather_kernel(idx_hbm, table_hbm, out_hbm, idx_v, rows_v, t5_v, *sems):
    sem_i = sems[:NBUF]
    sem_g = sems[NBUF:2 * NBUF]
    sem_w = sems[2 * NBUF:]
    wid = lax.axis_index("s") * NC + lax.axis_index("c")
    t0 = wid * BPW

    iota = lax.iota(jnp.int32, 16)
    mvecs = [(iota + k) & 15 for k in range(16)]
    msvecs = [(mv >> 3) * 4096 + (mv & 7) * 128 for mv in mvecs]

    def idx_desc(t, b):
        j = t // NCBG
        cbg = t % NCBG
        return pltpu.make_async_copy(
            idx_hbm.at[j, pl.ds(cbg * CR, CR)], idx_v.at[b], sem_i[b])

    def gather_desc(b):
        return pltpu.make_async_copy(
            table_hbm.at[idx_v.at[b]], rows_v.at[b], sem_g[b])

    def wb_descs(t, b):
        j = t // NCBG
        cbg = t % NCBG
        return [pltpu.make_async_copy(
                    t5_v.at[b, pl.ds(r * TW, TW)],
                    out_hbm.at[j, r, pl.ds(cbg * TW, TW)],
                    sem_w[b])
                for r in range(4)]

    def transpose_block(b):
        def sub(ti, carry):
            lb0 = ti * 16
            riota = iota + lb0
            for cb2 in (0, 16):
                s_sc = (lb0 // 128) * 1024 + (lb0 % 128) + (cb2 // 8) * TW
                sbase = iota + s_sc
                for k in range(16):
                    cv = mvecs[k] if cb2 == 0 else mvecs[k] + cb2
                    vv = plsc.load_gather(rows_v.at[b], [riota, cv])
                    plsc.store_scatter(t5_v.at[b], [msvecs[k] + sbase], vv)
            return carry

        lax.fori_loop(0, CR // 16, sub, 0)

    idx_desc(t0, 0).start()
    idx_desc(t0, 0).wait()
    gather_desc(0).start()
    idx_desc(t0 + 1, 1).start()

    def group(g_, carry):
        for b in range(NBUF):
            t = g_ * NBUF + b
            b1 = 1 - b

            @pl.when(t + 1 < BPW)
            def _():
                idx_desc(t0 + t + 1, b1).wait()
                gather_desc(b1).start()

            gather_desc(b).wait()

            @pl.when(t + 2 < BPW)
            def _():
                idx_desc(t0 + t + 2, b).start()

            @pl.when(t >= NBUF)
            def _():
                for d_ in wb_descs(t0 + t, b):
                    d_.wait()

            transpose_block(b)
            for d_ in wb_descs(t0 + t, b):
                d_.start()
        return carry

    lax.fori_loop(0, BPW // NBUF, group, 0)
    for b in range(NBUF):
        for d_ in wb_descs(t0 + BPW - NBUF + b, b):
            d_.wait()


NTCH = 7812
TPW = (NTCH + NW - 1) // NW


def _transpose_kernel(tT_hbm, tail_hbm, dense_hbm, buf0, buf1, stage0, stage1, *sems):
    bufs = (buf0, buf1)
    stages = (stage0, stage1)
    sem_i = sems[:2]
    sem_o = sems[2:]
    wid = lax.axis_index("s") * NC + lax.axis_index("c")
    c0 = wid * TPW

    iota = lax.iota(jnp.int32, 16)
    mvecs = [(iota + k) & 15 for k in range(16)]
    i32m = [iota * 32 + mv for mv in mvecs]

    def in_descs(c2, b):
        return [pltpu.make_async_copy(
                    tT_hbm.at[pl.ds(r * 8, 8), pl.ds(c2 * 128, 128)],
                    bufs[b].at[pl.ds(r * 8, 8), :], sem_i[b])
                for r in range(4)]

    def out_desc(c2, b):
        return pltpu.make_async_copy(
            stages[b], dense_hbm.at[pl.ds(c2 * 4096, 4096)], sem_o[b])

    def transpose_chunk(b):
        def rstep(ri, carry):
            r0 = ri * 16
            rvec = iota + r0
            for d0 in (0, 16):
                base = r0 * 32 + d0
                for k in range(16):
                    vv = plsc.load_gather(bufs[b], [mvecs[k] + d0, rvec])
                    plsc.store_scatter(stages[b], [i32m[k] + base], vv)
            return carry

        lax.fori_loop(0, 8, rstep, 0)

    @pl.when(c0 < NTCH)
    def _():
        for d_ in in_descs(c0, 0):
            d_.start()

    def group(g_, carry):
        for b in range(2):
            i = g_ * 2 + b
            b1 = 1 - b
            c2 = c0 + i

            @pl.when((i + 1 < TPW) & (c2 + 1 < NTCH))
            def _():
                for d_ in in_descs(c2 + 1, b1):
                    d_.start()

            @pl.when((i < TPW) & (c2 < NTCH))
            def _():
                for d_ in in_descs(c2, b):
                    d_.wait()

                @pl.when(i >= 2)
                def _():
                    out_desc(c2 - 2, b).wait()

                transpose_chunk(b)
                out_desc(c2, b).start()
        return carry

    lax.fori_loop(0, (TPW + 1) // 2, group, 0)
    for i in (TPW - 2, TPW - 1):
        c2 = c0 + i

        @pl.when(c2 < NTCH)
        def _():
            out_desc(c2, i % 2).wait()

    @pl.when(wid == 0)
    def _():
        pltpu.sync_copy(tail_hbm, stage0.at[pl.ds(0, 2048)])
        pltpu.sync_copy(stage0.at[pl.ds(0, 2048)],
                        dense_hbm.at[pl.ds(NTCH * 128 * D, 2048)])


@jax.jit
def _transpose_table(tT):
    mesh = plsc.VectorSubcoreMesh(core_axis_name="c", subcore_axis_name="s")
    f = pl.kernel(
        _transpose_kernel,
            out_type=jax.ShapeDtypeStruct((1000000 * D,), jnp.float32),
        mesh=mesh,
        scratch_types=(
            [pltpu.VMEM((D, 128), jnp.float32),
             pltpu.VMEM((D, 128), jnp.float32),
             pltpu.VMEM((4096,), jnp.float32),
             pltpu.VMEM((4096,), jnp.float32)]
            + [pltpu.SemaphoreType.DMA] * 4
        ),
        compiler_params=pltpu.CompilerParams(use_tc_tiling_on_sc=True,
                                             needs_layout_passes=False),
    )
    return f(tT[0], tT[1])


@jax.jit
def _gather(idxT, table):
    mesh = plsc.VectorSubcoreMesh(core_axis_name="c", subcore_axis_name="s")
    f = pl.kernel(
        _gather_kernel,
        out_type=jax.ShapeDtypeStruct((NJ, 4, NB * 8), jnp.float32),
        mesh=mesh,
        scratch_types=(
            [pltpu.VMEM((NBUF, CR), jnp.int32),
             pltpu.VMEM((NBUF, CR, D), jnp.float32),
             pltpu.VMEM((NBUF, 4 * TW), jnp.float32)]
            + [pltpu.SemaphoreType.DMA] * (3 * NBUF)
        ),
        compiler_params=pltpu.CompilerParams(use_tc_tiling_on_sc=False,
                                             needs_layout_passes=False),
    )
    return f(idxT, table)


def kernel(input, table):
    ntr = NTCH * 128
    tail = table[ntr:, :].reshape(-1)
    tflat = _transpose_table((table.T, tail))
    out5 = _gather(input.T, tflat.reshape(1000000, D))
    return (out5.reshape(NJ, 4, NB // 128, 8, 128)
            .transpose(2, 4, 0, 1, 3)
            .reshape(NB, NJ, D))


# speedup vs baseline: 2.0531x; 1.0098x over previous
import jax
import jax.numpy as jnp
from jax import lax
from jax.experimental import pallas as pl
from jax.experimental.pallas import tpu as pltpu
from jax.experimental.pallas import tpu_sc as plsc

NJ = 50
NB = 16384
D = 32
NC, NS = 2, 16
NW = NC * NS
G = 4
CR = G * 128
NCBG = NB // CR
NBLK = NJ * NCBG
BPW = NBLK // NW
TW = G * 1024
NBUF = 2


def _gather_kernel(idx_hbm, table_hbm, out_hbm, idx_v, rows_v, t5_v, *sems):
    sem_i = sems[:NBUF]
    sem_g = sems[NBUF:2 * NBUF]
    sem_w = sems[2 * NBUF:]
    wid = lax.axis_index("s") * NC + lax.axis_index("c")
    t0 = wid * BPW

    iota = lax.iota(jnp.int32, 16)
    mvecs = [(iota + k) & 15 for k in range(16)]
    msvecs = [(mv >> 3) * 4096 + (mv & 7) * 128 for mv in mvecs]

    def idx_desc(t, b):
        j = t // NCBG
        cbg = t % NCBG
        return pltpu.make_async_copy(
            idx_hbm.at[j, pl.ds(cbg * CR, CR)], idx_v.at[b], sem_i[b])

    def gather_desc(b):
        return pltpu.make_async_copy(
            table_hbm.at[idx_v.at[b]], rows_v.at[b], sem_g[b])

    def wb_descs(t, b):
        j = t // NCBG
        cbg = t % NCBG
        return [pltpu.make_async_copy(
                    t5_v.at[b, pl.ds(r * TW, TW)],
                    out_hbm.at[j, r, pl.ds(cbg * TW, TW)],
                    sem_w[b])
                for r in range(4)]

    def transpose_block(b):
        def sub(ti, carry):
            lb0 = ti * 16
            riota = iota + lb0
            for cb2 in (0, 16):
                s_sc = (lb0 // 128) * 1024 + (lb0 % 128) + (cb2 // 8) * TW
                sbase = iota + s_sc
                for k in range(16):
                    cv = mvecs[k] if cb2 == 0 else mvecs[k] + cb2
                    vv = plsc.load_gather(rows_v.at[b], [riota, cv])
                    plsc.store_scatter(t5_v.at[b], [msvecs[k] + sbase], vv)
            return carry

        lax.fori_loop(0, CR // 16, sub, 0)

    idx_desc(t0, 0).start()
    idx_desc(t0, 0).wait()
    gather_desc(0).start()
    idx_desc(t0 + 1, 1).start()

    def group(g_, carry):
        for b in range(NBUF):
            t = g_ * NBUF + b
            b1 = 1 - b

            @pl.when(t + 1 < BPW)
            def _():
                idx_desc(t0 + t + 1, b1).wait()
                gather_desc(b1).start()

            gather_desc(b).wait()

            @pl.when(t + 2 < BPW)
            def _():
                idx_desc(t0 + t + 2, b).start()

            @pl.when(t >= NBUF)
            def _():
                for d_ in wb_descs(t0 + t, b):
                    d_.wait()

            transpose_block(b)
            for d_ in wb_descs(t0 + t, b):
                d_.start()
        return carry

    lax.fori_loop(0, BPW // NBUF, group, 0)
    for b in range(NBUF):
        for d_ in wb_descs(t0 + BPW - NBUF + b, b):
            d_.wait()


NTCH = 7812
TPW = (NTCH + NW - 1) // NW


def _transpose_kernel(tT_hbm, tail_hbm, dense_hbm, buf0, buf1, stage0, stage1, *sems):
    bufs = (buf0, buf1)
    stages = (stage0, stage1)
    sem_i = sems[:2]
    sem_o = sems[2:]
    wid = lax.axis_index("s") * NC + lax.axis_index("c")
    c0 = wid * TPW

    iota = lax.iota(jnp.int32, 16)
    mvecs = [(iota + k) & 15 for k in range(16)]
    i32m = [iota * 32 + mv for mv in mvecs]

    def in_descs(c2, b):
        return [pltpu.make_async_copy(
                    tT_hbm.at[pl.ds(r * 8, 8), pl.ds(c2 * 128, 128)],
                    bufs[b].at[pl.ds(r * 8, 8), :], sem_i[b])
                for r in range(4)]

    def out_desc(c2, b):
        return pltpu.make_async_copy(
            stages[b], dense_hbm.at[pl.ds(c2 * 4096, 4096)], sem_o[b])

    def transpose_chunk(b):
        def rstep(ri, carry):
            for ru in range(2):
                r0 = (ri * 2 + ru) * 16
                rvec = iota + r0
                for d0 in (0, 16):
                    base = r0 * 32 + d0
                    for k in range(16):
                        vv = plsc.load_gather(bufs[b], [mvecs[k] + d0, rvec])
                        plsc.store_scatter(stages[b], [i32m[k] + base], vv)
            return carry

        lax.fori_loop(0, 4, rstep, 0)

    @pl.when(c0 < NTCH)
    def _():
        for d_ in in_descs(c0, 0):
            d_.start()

    def group(g_, carry):
        for b in range(2):
            i = g_ * 2 + b
            b1 = 1 - b
            c2 = c0 + i

            @pl.when((i + 1 < TPW) & (c2 + 1 < NTCH))
            def _():
                for d_ in in_descs(c2 + 1, b1):
                    d_.start()

            @pl.when((i < TPW) & (c2 < NTCH))
            def _():
                for d_ in in_descs(c2, b):
                    d_.wait()

                @pl.when(i >= 2)
                def _():
                    out_desc(c2 - 2, b).wait()

                transpose_chunk(b)
                out_desc(c2, b).start()
        return carry

    lax.fori_loop(0, (TPW + 1) // 2, group, 0)
    for i in (TPW - 2, TPW - 1):
        c2 = c0 + i

        @pl.when(c2 < NTCH)
        def _():
            out_desc(c2, i % 2).wait()

    @pl.when(wid == 0)
    def _():
        pltpu.sync_copy(tail_hbm, stage0.at[pl.ds(0, 2048)])
        pltpu.sync_copy(stage0.at[pl.ds(0, 2048)],
                        dense_hbm.at[pl.ds(NTCH * 128 * D, 2048)])


@jax.jit
def _transpose_table(tT):
    mesh = plsc.VectorSubcoreMesh(core_axis_name="c", subcore_axis_name="s")
    f = pl.kernel(
        _transpose_kernel,
            out_type=jax.ShapeDtypeStruct((1000000 * D,), jnp.float32),
        mesh=mesh,
        scratch_types=(
            [pltpu.VMEM((D, 128), jnp.float32),
             pltpu.VMEM((D, 128), jnp.float32),
             pltpu.VMEM((4096,), jnp.float32),
             pltpu.VMEM((4096,), jnp.float32)]
            + [pltpu.SemaphoreType.DMA] * 4
        ),
        compiler_params=pltpu.CompilerParams(use_tc_tiling_on_sc=True,
                                             needs_layout_passes=False),
    )
    return f(tT[0], tT[1])


@jax.jit
def _gather(idxT, table):
    mesh = plsc.VectorSubcoreMesh(core_axis_name="c", subcore_axis_name="s")
    f = pl.kernel(
        _gather_kernel,
        out_type=jax.ShapeDtypeStruct((NJ, 4, NB * 8), jnp.float32),
        mesh=mesh,
        scratch_types=(
            [pltpu.VMEM((NBUF, CR), jnp.int32),
             pltpu.VMEM((NBUF, CR, D), jnp.float32),
             pltpu.VMEM((NBUF, 4 * TW), jnp.float32)]
            + [pltpu.SemaphoreType.DMA] * (3 * NBUF)
        ),
        compiler_params=pltpu.CompilerParams(use_tc_tiling_on_sc=False,
                                             needs_layout_passes=False),
    )
    return f(idxT, table)


def kernel(input, table):
    ntr = NTCH * 128
    tail = table[ntr:, :].reshape(-1)
    tflat = _transpose_table((table.T, tail))
    out5 = _gather(input.T, tflat.reshape(1000000, D))
    return (out5.reshape(NJ, 4, NB // 128, 8, 128)
            .transpose(2, 4, 0, 1, 3)
            .reshape(NB, NJ, D))
